# baseline (speedup 1.0000x reference)
import jax
import jax.numpy as jnp
import numpy as np
from jax.experimental import pallas as pl
from jax.experimental.pallas import tpu as pltpu

DIM = 1024
HEADS = 16
HEAD_DIM = 64
WINDOW = 64
INNER = HEADS * HEAD_DIM
SCALE = HEAD_DIM ** (-0.5)
SEQ = 4096
NWIN = SEQ // WINDOW
J = 16
T = J * WINDOW
G = 1
PACK = G * WINDOW
NPACK = J // G


def _fused_kernel(x_ref, wqkv_ref, wout_ref, bout_ref, pos_ref, o_ref):
    xb = x_ref[0].reshape(T, DIM).astype(jnp.bfloat16)
    r1 = jax.lax.broadcasted_iota(jnp.int32, (T, T), 0)
    r2 = jax.lax.broadcasted_iota(jnp.int32, (T, T), 1)
    perm = (r2 == (r1 % WINDOW) * J + r1 // WINDOW).astype(jnp.bfloat16)
    xb = jnp.dot(perm, xb,
                 preferred_element_type=jnp.float32).astype(jnp.bfloat16)

    ii = jax.lax.broadcasted_iota(jnp.int32, (WINDOW, WINDOW), 0)
    jj = jax.lax.broadcasted_iota(jnp.int32, (WINDOW, WINDOW), 1)
    rel = jj - ii + (WINDOW - 1)
    kk = jax.lax.broadcasted_iota(jnp.int32, (WINDOW, WINDOW, 128), 2)
    onehot = (rel[:, :, None] == kk).astype(jnp.float32)
    bias = jnp.sum(onehot * pos_ref[0][None, None, :], axis=2)
    neg = jnp.full((WINDOW, WINDOW), -1e9, dtype=jnp.float32)
    biasmask = jnp.concatenate(
        [jnp.concatenate([bias if wc == wr else neg for wc in range(G)],
                         axis=1) for wr in range(G)], axis=0)

    qkv = jnp.dot(xb, wqkv_ref[...],
                  preferred_element_type=jnp.float32).astype(jnp.bfloat16)

    ones = jnp.ones((NPACK, PACK, HEAD_DIM), dtype=jnp.bfloat16)
    outs = []
    dn_qk = (((2,), (2,)), ((0,), (0,)))
    dn_pv = (((2,), (1,)), ((0,), (0,)))
    for h in range(HEADS):
        sl = slice(h * HEAD_DIM, (h + 1) * HEAD_DIM)
        qh = qkv[:, sl].reshape(NPACK, PACK, HEAD_DIM)
        kh = qkv[:, INNER + h * HEAD_DIM:INNER + (h + 1) * HEAD_DIM]
        kh = kh.reshape(NPACK, PACK, HEAD_DIM)
        vh = qkv[:, 2 * INNER + h * HEAD_DIM:2 * INNER + (h + 1) * HEAD_DIM]
        vh = vh.reshape(NPACK, PACK, HEAD_DIM)
        dots = jax.lax.dot_general(qh, kh, dn_qk,
                                   preferred_element_type=jnp.float32)
        dots = jnp.minimum(dots + biasmask[None, :, :], 80.0)
        p = jnp.exp(dots)
        vh_aug = jnp.concatenate([vh, ones], axis=2)
        oh = jax.lax.dot_general(p.astype(jnp.bfloat16), vh_aug, dn_pv,
                                 preferred_element_type=jnp.float32)
        oh = oh[:, :, :HEAD_DIM] / (oh[:, :, HEAD_DIM:] + 1e-30)
        outs.append(oh.reshape(T, HEAD_DIM).astype(jnp.bfloat16))

    attn_out = jnp.concatenate(outs, axis=1)
    out = jnp.dot(attn_out, wout_ref[...], preferred_element_type=jnp.float32)
    o_ref[0] = out + bout_ref[...]


def _run(x, W_qkv, W_out, b_out, pos_embedding):
    b, p, f = x.shape
    x4 = x.reshape(b, WINDOW, NWIN, f)
    colscale = jnp.concatenate([jnp.full((INNER,), SCALE, jnp.float32),
                                jnp.ones((2 * INNER,), jnp.float32)])
    wqkv = (W_qkv * colscale[None, :]).astype(jnp.bfloat16)
    wout = W_out.astype(jnp.bfloat16)
    bout = b_out.reshape(1, DIM)
    pos = jnp.pad(pos_embedding, (0, 1)).reshape(1, 128)

    grid = (b, NWIN // J)
    out = pl.pallas_call(
        _fused_kernel,
        grid=grid,
        in_specs=[
            pl.BlockSpec((1, WINDOW, J, DIM), lambda bi, ji: (bi, 0, ji, 0)),
            pl.BlockSpec((DIM, 3 * INNER), lambda bi, ji: (0, 0)),
            pl.BlockSpec((INNER, DIM), lambda bi, ji: (0, 0)),
            pl.BlockSpec((1, DIM), lambda bi, ji: (0, 0)),
            pl.BlockSpec((1, 128), lambda bi, ji: (0, 0)),
        ],
        out_specs=pl.BlockSpec((1, T, DIM), lambda bi, ji: (bi, ji, 0)),
        out_shape=jax.ShapeDtypeStruct((b, p, DIM), jnp.float32),
        compiler_params=pltpu.CompilerParams(
            dimension_semantics=("arbitrary", "arbitrary"),
        ),
    )(x4, wqkv, wout, bout, pos)
    return out


def kernel(x, W_qkv, W_out, b_out, pos_embedding):
    devs = jax.devices()
    if len(devs) >= 2 and x.shape[0] % 2 == 0:
        mesh = jax.sharding.Mesh(np.asarray(devs[:2]), ("d",))
        P = jax.sharding.PartitionSpec
        f = jax.shard_map(
            _run, mesh=mesh,
            in_specs=(P("d"), P(), P(), P(), P()),
            out_specs=P("d"),
            check_vma=False,
        )
        return f(x, W_qkv, W_out, b_out, pos_embedding)
    return _run(x, W_qkv, W_out, b_out, pos_embedding)


# speedup vs baseline: 3.8032x; 3.8032x over previous
import jax
import jax.numpy as jnp
import numpy as np
from jax.experimental import pallas as pl
from jax.experimental.pallas import tpu as pltpu

DIM = 1024
HEADS = 16
HEAD_DIM = 64
WINDOW = 64
INNER = HEADS * HEAD_DIM
SCALE = HEAD_DIM ** (-0.5)
SEQ = 4096
NWIN = SEQ // WINDOW
J = 16
T = J * WINDOW
G = 1
PACK = G * WINDOW
NPACK = J // G


def _fused_kernel(x_ref, wqkv_ref, wout_ref, bout_ref, pos_ref, o_ref):
    xb = x_ref[0].reshape(T, DIM).astype(jnp.bfloat16)
    r1 = jax.lax.broadcasted_iota(jnp.int32, (T, T), 0)
    r2 = jax.lax.broadcasted_iota(jnp.int32, (T, T), 1)
    perm = (r2 == (r1 % WINDOW) * J + r1 // WINDOW).astype(jnp.bfloat16)
    xb = jnp.dot(perm, xb,
                 preferred_element_type=jnp.float32).astype(jnp.bfloat16)

    ii = jax.lax.broadcasted_iota(jnp.int32, (WINDOW, WINDOW), 0)
    jj = jax.lax.broadcasted_iota(jnp.int32, (WINDOW, WINDOW), 1)
    rel = jj - ii + (WINDOW - 1)
    kk = jax.lax.broadcasted_iota(jnp.int32, (WINDOW, WINDOW, 128), 2)
    onehot = (rel[:, :, None] == kk).astype(jnp.float32)
    bias = jnp.sum(onehot * pos_ref[0][None, None, :], axis=2)
    neg = jnp.full((WINDOW, WINDOW), -1e9, dtype=jnp.float32)
    biasmask = jnp.concatenate(
        [jnp.concatenate([bias if wc == wr else neg for wc in range(G)],
                         axis=1) for wr in range(G)], axis=0)

    qkv = jnp.dot(xb, wqkv_ref[...],
                  preferred_element_type=jnp.float32).astype(jnp.bfloat16)

    ones = jnp.ones((NPACK, PACK, HEAD_DIM), dtype=jnp.bfloat16)
    outs = []
    dn_qk = (((2,), (2,)), ((0,), (0,)))
    dn_pv = (((2,), (1,)), ((0,), (0,)))
    for h in range(HEADS):
        sl = slice(h * HEAD_DIM, (h + 1) * HEAD_DIM)
        qh = qkv[:, sl].reshape(NPACK, PACK, HEAD_DIM)
        kh = qkv[:, INNER + h * HEAD_DIM:INNER + (h + 1) * HEAD_DIM]
        kh = kh.reshape(NPACK, PACK, HEAD_DIM)
        vh = qkv[:, 2 * INNER + h * HEAD_DIM:2 * INNER + (h + 1) * HEAD_DIM]
        vh = vh.reshape(NPACK, PACK, HEAD_DIM)
        dots = jax.lax.dot_general(qh, kh, dn_qk,
                                   preferred_element_type=jnp.float32)
        dots = jnp.minimum(dots + biasmask[None, :, :], 80.0)
        p = jnp.exp(dots)
        vh_aug = jnp.concatenate([vh, ones], axis=2)
        oh = jax.lax.dot_general(p.astype(jnp.bfloat16), vh_aug, dn_pv,
                                 preferred_element_type=jnp.float32)
        oh = oh[:, :, :HEAD_DIM] / (oh[:, :, HEAD_DIM:] + 1e-30)
        outs.append(oh.reshape(T, HEAD_DIM).astype(jnp.bfloat16))

    attn_out = jnp.concatenate(outs, axis=1)
    out = jnp.dot(attn_out, wout_ref[...], preferred_element_type=jnp.float32)
    o_ref[0] = out + bout_ref[...]


def _run(x, W_qkv, W_out, b_out, pos_embedding):
    b, p, f = x.shape
    x4 = x.reshape(b, WINDOW, NWIN, f)
    colscale = jnp.concatenate([jnp.full((INNER,), SCALE, jnp.float32),
                                jnp.ones((2 * INNER,), jnp.float32)])
    wqkv = (W_qkv * colscale[None, :]).astype(jnp.bfloat16)
    wout = W_out.astype(jnp.bfloat16)
    bout = b_out.reshape(1, DIM)
    pos = jnp.pad(pos_embedding, (0, 1)).reshape(1, 128)

    grid = (b, NWIN // J)
    out = pl.pallas_call(
        _fused_kernel,
        grid=grid,
        in_specs=[
            pl.BlockSpec((1, WINDOW, J, DIM), lambda bi, ji: (bi, 0, ji, 0)),
            pl.BlockSpec((DIM, 3 * INNER), lambda bi, ji: (0, 0)),
            pl.BlockSpec((INNER, DIM), lambda bi, ji: (0, 0)),
            pl.BlockSpec((1, DIM), lambda bi, ji: (0, 0)),
            pl.BlockSpec((1, 128), lambda bi, ji: (0, 0)),
        ],
        out_specs=pl.BlockSpec((1, T, DIM), lambda bi, ji: (bi, ji, 0)),
        out_shape=jax.ShapeDtypeStruct((b, p, DIM), jnp.float32),
        compiler_params=pltpu.CompilerParams(
            dimension_semantics=("arbitrary", "arbitrary"),
        ),
    )(x4, wqkv, wout, bout, pos)
    return out


def kernel(x, W_qkv, W_out, b_out, pos_embedding):
    return _run(x, W_qkv, W_out, b_out, pos_embedding)
